# Initial kernel scaffold; baseline (speedup 1.0000x reference)
import jax
import jax.numpy as jnp
from jax.experimental import pallas as pl


def kernel(output, target):
    raise NotImplementedError("write your pallas kernel here")


# speedup vs baseline: 2.3580x; 2.3580x over previous
import math

import jax
import jax.numpy as jnp
from jax.experimental import pallas as pl
from jax.experimental.pallas import tpu as pltpu

_V = 100000
_B = 1024
_EPS = 0.1
_PAD = 0
_S = _EPS / (_V - 2)
_C1 = (_V - 2) * _S * math.log(_S) + (1.0 - _EPS) * math.log(1.0 - _EPS)

_RBLK = 32
_NBLK = _B // _RBLK


def _body(t_ref, x_ref, o_ref, acc_ref):
    i = pl.program_id(0)

    @pl.when(i == 0)
    def _init():
        acc_ref[0, 0] = 0.0
        acc_ref[0, 1] = 0.0

    x = x_ref[...]
    t = t_ref[0]
    col = jax.lax.broadcasted_iota(jnp.int32, (_RBLK, _V), 1)
    w = jnp.where(col == t, 1.0 - _EPS,
                  jnp.where(col == _PAD, 0.0, _S))
    w = jnp.where(t == _PAD, 0.0, w)
    acc_ref[0, 0] += jnp.sum(w * x)
    acc_ref[0, 1] += jnp.sum((t != _PAD).astype(jnp.float32))

    @pl.when(i == _NBLK - 1)
    def _finish():
        o_ref[0, 0] = _C1 - acc_ref[0, 0] / acc_ref[0, 1]


def kernel(output, target):
    t3 = target.astype(jnp.int32).reshape(_NBLK, _RBLK, 1)
    res = pl.pallas_call(
        _body,
        grid=(_NBLK,),
        in_specs=[
            pl.BlockSpec((1, _RBLK, 1), lambda i: (i, 0, 0)),
            pl.BlockSpec((_RBLK, _V), lambda i: (i, 0)),
        ],
        out_specs=pl.BlockSpec((1, 1), lambda i: (0, 0),
                               memory_space=pltpu.SMEM),
        out_shape=jax.ShapeDtypeStruct((1, 1), jnp.float32),
        scratch_shapes=[pltpu.SMEM((1, 2), jnp.float32)],
        compiler_params=pltpu.CompilerParams(
            dimension_semantics=("arbitrary",),
        ),
    )(t3, output)
    return res[0, 0]


# speedup vs baseline: 2.3857x; 1.0117x over previous
import math

import jax
import jax.numpy as jnp
from jax.experimental import pallas as pl
from jax.experimental.pallas import tpu as pltpu

_V = 100000
_B = 1024
_EPS = 0.1
_PAD = 0
_S = _EPS / (_V - 2)
_C1 = (_V - 2) * _S * math.log(_S) + (1.0 - _EPS) * math.log(1.0 - _EPS)

_RBLK = 64
_NBLK = _B // _RBLK


def _partial_body(t_ref, x_ref, o_ref):
    x = x_ref[...]
    t = t_ref[0]
    col = jax.lax.broadcasted_iota(jnp.int32, (_RBLK, _V), 1)
    w = jnp.where(col == t, 1.0 - _EPS,
                  jnp.where(col == _PAD, 0.0, _S))
    w = jnp.where(t == _PAD, 0.0, w)
    o_ref[0, 0, 0] = jnp.sum(w * x)
    o_ref[0, 0, 1] = jnp.sum((t != _PAD).astype(jnp.float32))


def _combine_body(p_ref, o_ref):
    p = p_ref[...]
    o_ref[0, 0] = _C1 - jnp.sum(p[:, 0, 0]) / jnp.sum(p[:, 0, 1])


def kernel(output, target):
    t3 = target.astype(jnp.int32).reshape(_NBLK, _RBLK, 1)
    partials = pl.pallas_call(
        _partial_body,
        grid=(_NBLK,),
        in_specs=[
            pl.BlockSpec((1, _RBLK, 1), lambda i: (i, 0, 0)),
            pl.BlockSpec((_RBLK, _V), lambda i: (i, 0)),
        ],
        out_specs=pl.BlockSpec((1, 1, 2), lambda i: (i, 0, 0),
                               memory_space=pltpu.SMEM),
        out_shape=jax.ShapeDtypeStruct((_NBLK, 1, 2), jnp.float32),
        compiler_params=pltpu.CompilerParams(
            dimension_semantics=("parallel",),
        ),
    )(t3, output)
    res = pl.pallas_call(
        _combine_body,
        out_specs=pl.BlockSpec(memory_space=pltpu.SMEM),
        out_shape=jax.ShapeDtypeStruct((1, 1), jnp.float32),
    )(partials)
    return res[0, 0]


# speedup vs baseline: 2.3946x; 1.0037x over previous
import math

import jax
import jax.numpy as jnp
from jax.experimental import pallas as pl
from jax.experimental.pallas import tpu as pltpu

_V = 100000
_B = 1024
_EPS = 0.1
_PAD = 0
_S = _EPS / (_V - 2)
_C1 = (_V - 2) * _S * math.log(_S) + (1.0 - _EPS) * math.log(1.0 - _EPS)

_RBLK = 64
_NBLK = _B // _RBLK


_NSTRM = 4
_SRBLK = _RBLK // _NSTRM


def _partial_body(t_ref, *refs):
    xs = refs[:_NSTRM]
    o_ref = refs[_NSTRM]
    t = t_ref[0]
    col = jax.lax.broadcasted_iota(jnp.int32, (_SRBLK, _V), 1)
    acc = jnp.float32(0.0)
    for k in range(_NSTRM):
        tk = t[k * _SRBLK:(k + 1) * _SRBLK]
        w = jnp.where(col == tk, 1.0 - _EPS,
                      jnp.where(col == _PAD, 0.0, _S))
        w = jnp.where(tk == _PAD, 0.0, w)
        acc += jnp.sum(w * xs[k][...])
    o_ref[0, 0, 0] = acc
    o_ref[0, 0, 1] = jnp.sum((t != _PAD).astype(jnp.float32))


def _combine_body(p_ref, o_ref):
    p = p_ref[...]
    o_ref[0, 0] = _C1 - jnp.sum(p[:, 0, 0]) / jnp.sum(p[:, 0, 1])


def kernel(output, target):
    t3 = target.astype(jnp.int32).reshape(_NBLK, _RBLK, 1)
    partials = pl.pallas_call(
        _partial_body,
        grid=(_NBLK,),
        in_specs=[pl.BlockSpec((1, _RBLK, 1), lambda i: (i, 0, 0))] + [
            pl.BlockSpec((_SRBLK, _V),
                         lambda i, k=k: (i * _NSTRM + k, 0))
            for k in range(_NSTRM)
        ],
        out_specs=pl.BlockSpec((1, 1, 2), lambda i: (i, 0, 0),
                               memory_space=pltpu.SMEM),
        out_shape=jax.ShapeDtypeStruct((_NBLK, 1, 2), jnp.float32),
        compiler_params=pltpu.CompilerParams(
            dimension_semantics=("parallel",),
        ),
    )(t3, *([output] * _NSTRM))
    res = pl.pallas_call(
        _combine_body,
        out_specs=pl.BlockSpec(memory_space=pltpu.SMEM),
        out_shape=jax.ShapeDtypeStruct((1, 1), jnp.float32),
    )(partials)
    return res[0, 0]
